# baseline (speedup 1.0000x reference)
import jax
import jax.numpy as jnp
from jax.experimental import pallas as pl

N = 50000
E = 800000
G = 16
NELEM = 10
C = 64
NB = 8
RMAX = 5.0
P = 5
AVG_NEIGH = 16.0


def _sph(unit):
    x, y, z = unit[:, 0], unit[:, 1], unit[:, 2]
    s3 = jnp.sqrt(3.0)
    s15 = jnp.sqrt(15.0)
    s5 = jnp.sqrt(5.0)
    return jnp.stack([
        jnp.ones_like(x),
        s3 * x, s3 * y, s3 * z,
        s15 * x * y, s15 * y * z,
        (s5 / 2.0) * (3.0 * z * z - 1.0),
        s15 * x * z,
        (s15 / 2.0) * (x * x - y * y),
    ], axis=-1)


def kernel(node_attrs, positions, edge_index, shifts, batch, ptr, total_charge, params):
    sender, receiver = edge_index[0], edge_index[1]
    vectors = positions[receiver] - positions[sender] + shifts
    lengths = jnp.sqrt(jnp.sum(vectors * vectors, axis=-1, keepdims=True) + 1e-12)
    unit = vectors / lengths
    Y = _sph(unit)
    nf = jnp.arange(1, NB + 1, dtype=jnp.float32)
    bess = jnp.sqrt(2.0 / RMAX) * jnp.sin(nf * jnp.pi * lengths / RMAX) / lengths
    u = lengths / RMAX
    env = (1.0 - (P + 1.0) * (P + 2.0) / 2.0 * u ** P
           + P * (P + 2.0) * u ** (P + 1) - P * (P + 1.0) / 2.0 * u ** (P + 2))
    env = env * (u < 1.0)
    edge_feats = bess * env
    feats = jnp.zeros((N, C, 9), dtype=jnp.float32).at[:, :, 0].set(node_attrs @ params["W_embed"])
    charges, dipoles, polars = [], [], []
    for i in range(2):
        p = params["layer" + str(i)]
        R = jax.nn.silu(edge_feats @ p["Wr1"])
        R = jax.nn.silu(R @ p["Wr2"])
        R = R @ p["Wr3"]
        s = feats[:, :, 0]
        msg = (R * s[sender])[:, :, None] * Y[:, None, :]
        agg = jax.ops.segment_sum(msg, receiver, num_segments=N) / AVG_NEIGH
        agg = jnp.concatenate([
            jnp.einsum("ncd,ce->ned", agg[:, :, 0:1], p["Wmix0"]),
            jnp.einsum("ncd,ce->ned", agg[:, :, 1:4], p["Wmix1"]),
            jnp.einsum("ncd,ce->ned", agg[:, :, 4:9], p["Wmix2"]),
        ], axis=-1)
        gate = jax.nn.silu(agg[:, :, 0] @ p["Wgate"])
        feats = agg * gate[:, :, None] + feats
        s2 = feats[:, :, 0]
        q = s2 @ p["wq"]
        iso = s2 @ p["wiso"]
        dip = jnp.einsum("ncd,c->nd", feats[:, :, 1:4], p["wd"])
        pol5 = jnp.einsum("ncd,c->nd", feats[:, :, 4:9], p["wp"])
        charges.append(q)
        dipoles.append(dip)
        polars.append(jnp.concatenate([iso[:, None], pol5], axis=-1))
    atomic_charges = jnp.stack(charges, -1).sum(-1)
    atomic_dipoles = jnp.stack(dipoles, -1).sum(-1)
    atomic_polar = jnp.stack(polars, -1).sum(-1)
    num_atoms = (ptr[1:] - ptr[:-1]).astype(jnp.float32)
    mean_q = jax.ops.segment_sum(atomic_charges, batch, num_segments=G) / num_atoms
    excess = mean_q - total_charge / num_atoms
    atomic_charges = atomic_charges - excess[batch]
    total_dipole = jax.ops.segment_sum(atomic_dipoles, batch, num_segments=G)
    baseline = jax.ops.segment_sum(atomic_charges[:, None] * positions, batch, num_segments=G)
    total_dipole = total_dipole + baseline
    pol6 = jax.ops.segment_sum(atomic_polar, batch, num_segments=G)
    s2_, s3_, s6_ = jnp.sqrt(2.0), jnp.sqrt(3.0), jnp.sqrt(6.0)
    iso_g = pol6[:, 0] / s3_
    xy = pol6[:, 1] / s2_
    yz = pol6[:, 2] / s2_
    z2 = pol6[:, 3]
    xz = pol6[:, 4] / s2_
    x2y2 = pol6[:, 5] / s2_
    A00 = iso_g - z2 / s6_ + x2y2
    A11 = iso_g - z2 / s6_ - x2y2
    A22 = iso_g + 2.0 * z2 / s6_
    A = jnp.stack([
        jnp.stack([A00, xy, xz], -1),
        jnp.stack([xy, A11, yz], -1),
        jnp.stack([xz, yz, A22], -1),
    ], axis=1)
    return jnp.concatenate([total_dipole, A.reshape(G, 9)], axis=-1)


# speedup vs baseline: 6.3818x; 6.3818x over previous
import functools
import jax
import jax.numpy as jnp
from jax import lax
from jax.experimental import pallas as pl
from jax.experimental.pallas import tpu as pltpu
from jax.experimental.pallas import tpu_sc as plsc

N = 50000
E = 800000
G = 16
C = 64
RMAX = 5.0
AVG_NEIGH = 16.0

NP = 50176
EP = 802816
NBLK = EP // 128
BPW_A = NBLK // 32
BPT_B = NBLK // 16
HALF = 25000
HALFP = 25008
ZR = HALFP // 16
DUMP = 25000
GSZ = N // G

_mesh = plsc.VectorSubcoreMesh(core_axis_name="c", subcore_axis_name="s")
_sc_params = pltpu.CompilerParams(use_tc_tiling_on_sc=False,
                                  needs_layout_passes=False)


@functools.partial(
    pl.kernel, mesh=_mesh, compiler_params=_sc_params,
    out_type=jax.ShapeDtypeStruct((EP, 16), jnp.float32),
    scratch_types=[
        pltpu.VMEM((128,), jnp.int32),
        pltpu.VMEM((128, 16), jnp.float32),
        pltpu.VMEM((128, 16), jnp.float32),
        pltpu.SemaphoreType.DMA,
    ],
)
def _sc_vec(pos_hbm, snd_hbm, rcv_hbm, out_hbm, idx_v, a_v, b_v, sem):
    c = lax.axis_index("c")
    s = lax.axis_index("s")
    wid = s * 2 + c

    def body(b, carry):
        blk = wid * BPW_A + b
        pltpu.sync_copy(snd_hbm.at[blk], idx_v)
        pltpu.async_copy(pos_hbm.at[idx_v], a_v, sem).wait()
        pltpu.sync_copy(rcv_hbm.at[blk], idx_v)
        pltpu.async_copy(pos_hbm.at[idx_v], b_v, sem).wait()

        def rb(j, cr):
            b_v[j, :] = b_v[j, :] - a_v[j, :]
            return cr

        lax.fori_loop(0, 128, rb, 0)
        pltpu.sync_copy(b_v, out_hbm.at[pl.ds(blk * 128, 128), :])
        return carry

    lax.fori_loop(0, BPW_A, body, 0)


@functools.partial(
    pl.kernel, mesh=_mesh, compiler_params=_sc_params,
    out_type=jax.ShapeDtypeStruct((EP, 64), jnp.float32),
    scratch_types=[
        pltpu.VMEM((128,), jnp.int32),
        pltpu.VMEM((128, 64), jnp.float32),
        pltpu.VMEM((128, 64), jnp.float32),
        pltpu.SemaphoreType.DMA,
    ],
)
def _sc_rs(s_hbm, snd_hbm, r_hbm, out_hbm, idx_v, srow_v, r_v, sem):
    c = lax.axis_index("c")
    s = lax.axis_index("s")
    wid = s * 2 + c

    def body(b, carry):
        blk = wid * BPW_A + b
        pltpu.sync_copy(snd_hbm.at[blk], idx_v)
        pltpu.async_copy(s_hbm.at[idx_v], srow_v, sem).wait()
        pltpu.sync_copy(r_hbm.at[pl.ds(blk * 128, 128), :], r_v)

        def rb(j, cr):
            for q in range(4):
                sl = pl.ds(q * 16, 16)
                srow_v[j, sl] = srow_v[j, sl] * r_v[j, sl]
            return cr

        lax.fori_loop(0, 128, rb, 0)
        pltpu.sync_copy(srow_v, out_hbm.at[pl.ds(blk * 128, 128), :])
        return carry

    lax.fori_loop(0, BPW_A, body, 0)


@functools.partial(
    pl.kernel, mesh=_mesh, compiler_params=_sc_params,
    out_type=jax.ShapeDtypeStruct((2, HALFP, 64), jnp.float32),
    scratch_types=[
        pltpu.VMEM((128, 64), jnp.float32),
        pltpu.VMEM((128,), jnp.int32),
        pltpu.VMEM((128,), jnp.float32),
        pltpu.VMEM_SHARED((HALFP, 64), jnp.float32),
    ],
)
def _sc_scat(rs_hbm, yd_hbm, idx_hbm, z_hbm, out_hbm, rs_v, idx_v, yd_v, acc):
    c = lax.axis_index("c")
    s = lax.axis_index("s")
    pltpu.sync_copy(z_hbm, acc.at[pl.ds(s * ZR, ZR), :])
    plsc.subcore_barrier()

    def body(b, carry):
        blk = s * BPT_B + b
        pltpu.sync_copy(rs_hbm.at[pl.ds(blk * 128, 128), :], rs_v)
        pltpu.sync_copy(idx_hbm.at[c, blk], idx_v)
        pltpu.sync_copy(yd_hbm.at[pl.ds(blk * 128, 128)], yd_v)

        def rb(j, cr):
            yb = plsc.load_gather(yd_v, [jnp.full((16,), j, jnp.int32)])
            for q in range(4):
                sl = pl.ds(q * 16, 16)
                rs_v[j, sl] = rs_v[j, sl] * yb
            return cr

        lax.fori_loop(0, 128, rb, 0)
        pltpu.sync_copy(rs_v, acc.at[idx_v], add=True)
        return carry

    lax.fori_loop(0, BPT_B, body, 0)
    plsc.subcore_barrier()
    pltpu.sync_copy(acc.at[pl.ds(s * ZR, ZR), :],
                    out_hbm.at[c, pl.ds(s * ZR, ZR), :])


def _silu(x):
    return x / (1.0 + jnp.exp(-x))


def _tc_embed_body(na_ref, w_ref, o_ref):
    o_ref[...] = jnp.dot(na_ref[...], w_ref[...],
                         preferred_element_type=jnp.float32)


def _tc_embed(na, w):
    return pl.pallas_call(
        _tc_embed_body,
        grid=(NP // 512,),
        in_specs=[
            pl.BlockSpec((512, na.shape[1]), lambda i: (i, 0)),
            pl.BlockSpec(w.shape, lambda i: (0, 0)),
        ],
        out_specs=pl.BlockSpec((512, 64), lambda i: (i, 0)),
        out_shape=jax.ShapeDtypeStruct((NP, 64), jnp.float32),
    )(na, w)


def _tc_edge_body(vec_ref, rcv_ref, w01, w02, w03, w11, w12, w13,
                  y_ref, r0_ref, r1_ref, i0_ref, i1_ref):
    v = vec_ref[...]
    l2 = jnp.sum(v[:, 0:3] * v[:, 0:3], axis=1, keepdims=True) + 1e-12
    length = jnp.sqrt(l2)
    inv = 1.0 / length
    x = v[:, 0:1] * inv
    y = v[:, 1:2] * inv
    z = v[:, 2:3] * inv
    s3 = jnp.sqrt(3.0)
    s15 = jnp.sqrt(15.0)
    s5 = jnp.sqrt(5.0)
    one = jnp.ones_like(x)
    ycols = jnp.concatenate([
        one, s3 * x, s3 * y, s3 * z,
        s15 * x * y, s15 * y * z,
        (s5 / 2.0) * (3.0 * z * z - 1.0),
        s15 * x * z, (s15 / 2.0) * (x * x - y * y),
        jnp.zeros((v.shape[0], 7), jnp.float32),
    ], axis=1)
    y_ref[...] = ycols

    nf = lax.broadcasted_iota(jnp.int32, (v.shape[0], 8), 1).astype(
        jnp.float32) + 1.0
    bess = jnp.sqrt(2.0 / RMAX) * jnp.sin(nf * (jnp.pi / RMAX) * length) * inv
    u = length / RMAX
    u2 = u * u
    u4 = u2 * u2
    u5 = u4 * u
    u6 = u5 * u
    u7 = u6 * u
    env = 1.0 - 21.0 * u5 + 35.0 * u6 - 15.0 * u7
    env = jnp.where(u < 1.0, env, 0.0)
    ef = bess * env

    def mlp(wa, wb, wc):
        h = _silu(jnp.dot(ef, wa[...], preferred_element_type=jnp.float32))
        h = _silu(jnp.dot(h, wb[...], preferred_element_type=jnp.float32))
        return jnp.dot(h, wc[...],
                       preferred_element_type=jnp.float32) * (1.0 / AVG_NEIGH)

    r0_ref[...] = mlp(w01, w02, w03)
    r1_ref[...] = mlp(w11, w12, w13)

    r = rcv_ref[...]
    i0_ref[...] = jnp.where(r < HALF, r, DUMP)
    i1_ref[...] = jnp.where(r >= HALF, r - HALF, DUMP)


def _tc_edge(vec, rcv, p0, p1):
    wsp = lambda shp: pl.BlockSpec(shp, lambda i: (0, 0))
    return pl.pallas_call(
        _tc_edge_body,
        grid=(EP // 512,),
        in_specs=[
            pl.BlockSpec((512, 16), lambda i: (i, 0)),
            pl.BlockSpec((512,), lambda i: (i,)),
            wsp((8, 64)), wsp((64, 64)), wsp((64, 64)),
            wsp((8, 64)), wsp((64, 64)), wsp((64, 64)),
        ],
        out_specs=[
            pl.BlockSpec((512, 16), lambda i: (i, 0)),
            pl.BlockSpec((512, 64), lambda i: (i, 0)),
            pl.BlockSpec((512, 64), lambda i: (i, 0)),
            pl.BlockSpec((512,), lambda i: (i,)),
            pl.BlockSpec((512,), lambda i: (i,)),
        ],
        out_shape=[
            jax.ShapeDtypeStruct((EP, 16), jnp.float32),
            jax.ShapeDtypeStruct((EP, 64), jnp.float32),
            jax.ShapeDtypeStruct((EP, 64), jnp.float32),
            jax.ShapeDtypeStruct((EP,), jnp.int32),
            jax.ShapeDtypeStruct((EP,), jnp.int32),
        ],
    )(vec, rcv, p0["Wr1"], p0["Wr2"], p0["Wr3"],
      p1["Wr1"], p1["Wr2"], p1["Wr3"])


def _tc_node_body(agg_ref, f_ref, m0, m1, m2, wg, wp_ref, fo_ref, pr_ref):
    gate_in = jnp.dot(agg_ref[0], m0[...], preferred_element_type=jnp.float32)
    gate = _silu(jnp.dot(gate_in, wg[...], preferred_element_type=jnp.float32))
    proj = jnp.zeros((agg_ref.shape[1], 16), jnp.float32)
    for d in range(9):
        w = m0 if d == 0 else (m1 if d < 4 else m2)
        mixed = jnp.dot(agg_ref[d], w[...], preferred_element_type=jnp.float32)
        fnew = mixed * gate + f_ref[d]
        fo_ref[d] = fnew
        proj = proj + jnp.dot(fnew, wp_ref[d],
                              preferred_element_type=jnp.float32)
    pr_ref[...] = proj


def _tc_node(agg, feats, p, wproj):
    wsp = lambda shp: pl.BlockSpec(shp, lambda i: (0, 0))
    return pl.pallas_call(
        _tc_node_body,
        grid=(NP // 512,),
        in_specs=[
            pl.BlockSpec((9, 512, 64), lambda i: (0, i, 0)),
            pl.BlockSpec((9, 512, 64), lambda i: (0, i, 0)),
            wsp((64, 64)), wsp((64, 64)), wsp((64, 64)), wsp((64, 64)),
            pl.BlockSpec((9, 64, 16), lambda i: (0, 0, 0)),
        ],
        out_specs=[
            pl.BlockSpec((9, 512, 64), lambda i: (0, i, 0)),
            pl.BlockSpec((512, 16), lambda i: (i, 0)),
        ],
        out_shape=[
            jax.ShapeDtypeStruct((9, NP, 64), jnp.float32),
            jax.ShapeDtypeStruct((NP, 16), jnp.float32),
        ],
    )(agg, feats, p["Wmix0"], p["Wmix1"], p["Wmix2"], p["Wgate"], wproj)


def _tc_final_body(p0_ref, p1_ref, pos_ref, o_ref):
    p = p0_ref[...] + p1_ref[...]
    sums = jnp.sum(p, axis=1)
    q = p[:, :, 0]
    excess = sums[:, 0:1] / float(GSZ)
    qc = q - excess
    pos = pos_ref[...]
    bl = jnp.sum(qc[:, :, None] * pos, axis=1)
    dip = sums[:, 2:5] + bl[:, 0:3]
    s2 = jnp.sqrt(2.0)
    s3 = jnp.sqrt(3.0)
    s6 = jnp.sqrt(6.0)
    iso = sums[:, 1:2] / s3
    xy = sums[:, 5:6] / s2
    yz = sums[:, 6:7] / s2
    z2 = sums[:, 7:8]
    xz = sums[:, 8:9] / s2
    x2y2 = sums[:, 9:10] / s2
    a00 = iso - z2 / s6 + x2y2
    a11 = iso - z2 / s6 - x2y2
    a22 = iso + 2.0 * z2 / s6
    row = jnp.concatenate(
        [dip, a00, xy, xz, xy, a11, yz, xz, yz, a22,
         jnp.zeros((1, 4), jnp.float32)], axis=1)
    o_ref[...] = row.reshape(1, 1, 16)


def _tc_final(p0, p1, pos):
    out = pl.pallas_call(
        _tc_final_body,
        grid=(G,),
        in_specs=[pl.BlockSpec((1, GSZ, 16), lambda i: (i, 0, 0))] * 3,
        out_specs=pl.BlockSpec((1, 1, 16), lambda i: (i, 0, 0)),
        out_shape=jax.ShapeDtypeStruct((G, 1, 16), jnp.float32),
    )(p0, p1, pos)
    return out.reshape(G, 16)


def _build_wproj(p):
    w = jnp.zeros((9, 64, 16), jnp.float32)
    w = w.at[0, :, 0].set(p["wq"])
    w = w.at[0, :, 1].set(p["wiso"])
    for d in range(3):
        w = w.at[1 + d, :, 2 + d].set(p["wd"])
    for d in range(5):
        w = w.at[4 + d, :, 5 + d].set(p["wp"])
    return w


def kernel(node_attrs, positions, edge_index, shifts, batch, ptr,
           total_charge, params):
    snd = edge_index[0].astype(jnp.int32)
    rcv = edge_index[1].astype(jnp.int32)
    pad_e = EP - E
    snd_p = jnp.concatenate([snd, jnp.zeros((pad_e,), jnp.int32)])
    rcv_p = jnp.concatenate([rcv, jnp.full((pad_e,), N, jnp.int32)])
    snd_b = snd_p.reshape(NBLK, 128)
    rcv_b = rcv_p.reshape(NBLK, 128)

    pos16 = jnp.zeros((NP, 16), jnp.float32).at[:N, :3].set(positions)
    na_p = jnp.zeros((NP, node_attrs.shape[1]),
                     jnp.float32).at[:N].set(node_attrs)

    s0 = _tc_embed(na_p, params["W_embed"])

    vec = _sc_vec(pos16, snd_b, rcv_b)
    yc, r0, r1, i0, i1 = _tc_edge(vec, rcv_p, params["layer0"],
                                  params["layer1"])
    idx3 = jnp.stack([i0.reshape(NBLK, 128), i1.reshape(NBLK, 128)])
    zslab = jnp.zeros((ZR, 64), jnp.float32)

    feats = jnp.concatenate(
        [s0[None], jnp.zeros((8, NP, 64), jnp.float32)], axis=0)
    projs = []
    for li, (rr, p) in enumerate([(r0, params["layer0"]),
                                  (r1, params["layer1"])]):
        stab = feats[0]
        rs = _sc_rs(stab, snd_b, rr)
        aggs = []
        for d in range(9):
            out = _sc_scat(rs, yc[:, d], idx3, zslab)
            aggs.append(out[:, :HALF, :].reshape(N, 64))
        agg = jnp.stack(aggs)
        agg = jnp.concatenate(
            [agg, jnp.zeros((9, NP - N, 64), jnp.float32)], axis=1)
        feats, proj = _tc_node(agg, feats, p, _build_wproj(p))
        projs.append(proj)

    p0 = projs[0][:N].reshape(G, GSZ, 16)
    p1 = projs[1][:N].reshape(G, GSZ, 16)
    posr = pos16[:N].reshape(G, GSZ, 16)
    out = _tc_final(p0, p1, posr)
    return out[:, :12]
